# Initial kernel scaffold; baseline (speedup 1.0000x reference)
import jax
import jax.numpy as jnp
from jax.experimental import pallas as pl


def kernel(x, W):
    raise NotImplementedError("write your pallas kernel here")


# speedup vs baseline: 4.5573x; 4.5573x over previous
import functools

import jax
import jax.numpy as jnp
from jax import lax
from jax.experimental import pallas as pl

_CAPACITY_FACTOR = 1.25


def _router_body(x_ref, w_ref, sc_ref, rw_ref, ei_ref, *,
                 nt, e, s, b, cap, tblk, ch):
    i = pl.program_id(0)
    nb = pl.num_programs(0)
    lb = lax.dot_general(x_ref[...], w_ref[...], (((1,), (1,)), ((), ())),
                         preferred_element_type=jnp.float32)
    sc_ref[pl.ds(i * tblk, tblk), :] = lb

    @pl.when(i == nb - 1)
    def _final():
        nch_s = s // ch
        nch_t = nt // ch

        for bb in range(b):
            base = bb * s

            def mx_body(c, m):
                blk = sc_ref[pl.ds(base + c * ch, ch), :]
                return jnp.maximum(m, jnp.max(blk, axis=0, keepdims=True))
            m = lax.fori_loop(0, nch_s, mx_body,
                              jnp.full((1, e), -jnp.inf, jnp.float32))

            def ex_body(c, z):
                idx = base + c * ch
                eb = jnp.exp(sc_ref[pl.ds(idx, ch), :] - m)
                sc_ref[pl.ds(idx, ch), :] = eb
                return z + jnp.sum(eb, axis=0, keepdims=True)
            z = lax.fori_loop(0, nch_s, ex_body, jnp.zeros((1, e), jnp.float32))

            def nm_body(c, carry):
                idx = base + c * ch
                sc_ref[pl.ds(idx, ch), :] = sc_ref[pl.ds(idx, ch), :] / z
                return carry
            lax.fori_loop(0, nch_s, nm_body, 0)

        def cnt_ge(th):
            def body(c, acc):
                bi = lax.bitcast_convert_type(
                    sc_ref[pl.ds(c * ch, ch), :], jnp.int32)
                return acc + jnp.sum(jnp.where(bi >= th, 1, 0),
                                     axis=0, keepdims=True)
            return lax.fori_loop(0, nch_t, body, jnp.zeros((1, e), jnp.int32))

        def bs_body(_, lh):
            lo, hi = lh
            mid = lo + lax.shift_right_logical(hi - lo, 1)
            p = cnt_ge(mid) >= cap
            return jnp.where(p, mid, lo), jnp.where(p, hi, mid)
        tau, _ = lax.fori_loop(0, 31, bs_body,
                               (jnp.zeros((1, e), jnp.int32),
                                jnp.full((1, e), 0x7F800000, jnp.int32)))
        r = cap - cnt_ge(tau + 1)

        def cnt_eq_lt(bound):
            def body(c, acc):
                bi = lax.bitcast_convert_type(
                    sc_ref[pl.ds(c * ch, ch), :], jnp.int32)
                idx = lax.broadcasted_iota(jnp.int32, (ch, e), 0) + c * ch
                mask = (bi == tau) & (idx < bound)
                return acc + jnp.sum(jnp.where(mask, 1, 0),
                                     axis=0, keepdims=True)
            return lax.fori_loop(0, nch_t, body, jnp.zeros((1, e), jnp.int32))

        def ib_body(_, lh):
            lo2, hi2 = lh
            mid = lo2 + lax.shift_right_logical(hi2 - lo2, 1)
            p = cnt_eq_lt(mid) >= r
            return jnp.where(p, lo2, mid), jnp.where(p, mid, hi2)
        _, ibound = lax.fori_loop(0, 15, ib_body,
                                  (jnp.zeros((1, e), jnp.int32),
                                   jnp.full((1, e), nt, jnp.int32)))

        iota_e = lax.broadcasted_iota(jnp.int32, (ch, e), 1)

        def sel_body(c, carry):
            blk = sc_ref[pl.ds(c * ch, ch), :]
            bi = lax.bitcast_convert_type(blk, jnp.int32)
            idx = lax.broadcasted_iota(jnp.int32, (ch, e), 0) + c * ch
            sel = (bi > tau) | ((bi == tau) & (idx < ibound))
            emax = jnp.max(jnp.where(sel, iota_e, -1), axis=1, keepdims=True)
            w = jnp.sum(jnp.where(iota_e == emax, blk, 0.0),
                        axis=1, keepdims=True)
            ei_ref[pl.ds(c * ch, ch), :] = jnp.maximum(emax, 0)
            rw_ref[pl.ds(c * ch, ch), :] = w
            return carry
        lax.fori_loop(0, nch_t, sel_body, 0)


def kernel(x, W):
    b, s, d = x.shape
    e = W.shape[0]
    nt = b * s
    cap = min(int(_CAPACITY_FACTOR * nt / e), nt)
    tblk = 512
    ch = 2048
    xf = x.reshape(nt, d)
    body = functools.partial(_router_body, nt=nt, e=e, s=s, b=b,
                             cap=cap, tblk=tblk, ch=ch)
    scores, rw, ei = pl.pallas_call(
        body,
        grid=(nt // tblk,),
        in_specs=[
            pl.BlockSpec((tblk, d), lambda i: (i, 0)),
            pl.BlockSpec((e, d), lambda i: (0, 0)),
        ],
        out_specs=[
            pl.BlockSpec((nt, e), lambda i: (0, 0)),
            pl.BlockSpec((nt, 1), lambda i: (0, 0)),
            pl.BlockSpec((nt, 1), lambda i: (0, 0)),
        ],
        out_shape=[
            jax.ShapeDtypeStruct((nt, e), jnp.float32),
            jax.ShapeDtypeStruct((nt, 1), jnp.float32),
            jax.ShapeDtypeStruct((nt, 1), jnp.int32),
        ],
    )(xf, W)
    return (rw.reshape(b, s, 1), ei.reshape(b, s, 1), scores.reshape(b, s, e))


# speedup vs baseline: 5.5518x; 1.2182x over previous
import functools

import jax
import jax.numpy as jnp
from jax import lax
from jax.experimental import pallas as pl
from jax.experimental.pallas import tpu as pltpu
from jax.experimental.pallas import tpu_sc as plsc

_CAPACITY_FACTOR = 1.25
_NT = 32768
_E = 64
_CAP = 640
_BINS = 512
_NV = _NT // 16


def _tc_body(x_ref, w_ref, sc_ref, sct_ref, *, nt, e, s, b, tblk, ch):
    i = pl.program_id(0)
    nb = pl.num_programs(0)
    lb = lax.dot_general(x_ref[...], w_ref[...], (((1,), (1,)), ((), ())),
                         preferred_element_type=jnp.float32)
    sc_ref[pl.ds(i * tblk, tblk), :] = lb

    @pl.when(i == nb - 1)
    def _final():
        nch_s = s // ch
        for bb in range(b):
            base = bb * s

            def mx_body(cc, m):
                blk = sc_ref[pl.ds(base + cc * ch, ch), :]
                return jnp.maximum(m, jnp.max(blk, axis=0, keepdims=True))
            m = lax.fori_loop(0, nch_s, mx_body,
                              jnp.full((1, e), -jnp.inf, jnp.float32))

            def ex_body(cc, z):
                idx = base + cc * ch
                eb = jnp.exp(sc_ref[pl.ds(idx, ch), :] - m)
                sc_ref[pl.ds(idx, ch), :] = eb
                return z + jnp.sum(eb, axis=0, keepdims=True)
            z = lax.fori_loop(0, nch_s, ex_body, jnp.zeros((1, e), jnp.float32))

            def nm_body(cc, carry):
                idx = base + cc * ch
                blk = sc_ref[pl.ds(idx, ch), :] / z
                sc_ref[pl.ds(idx, ch), :] = blk
                sct_ref[:, pl.ds(idx, ch)] = jnp.swapaxes(blk, 0, 1)
                return carry
            lax.fori_loop(0, nch_s, nm_body, 0)


def _tc_scores(xf, W, nt, e, s, b):
    tblk, ch = 512, 2048
    body = functools.partial(_tc_body, nt=nt, e=e, s=s, b=b, tblk=tblk, ch=ch)
    return pl.pallas_call(
        body,
        grid=(nt // tblk,),
        in_specs=[
            pl.BlockSpec((tblk, 768), lambda i: (i, 0)),
            pl.BlockSpec((e, 768), lambda i: (0, 0)),
        ],
        out_specs=[
            pl.BlockSpec((nt, e), lambda i: (0, 0)),
            pl.BlockSpec((e, nt), lambda i: (0, 0)),
        ],
        out_shape=[
            jax.ShapeDtypeStruct((nt, e), jnp.float32),
            jax.ShapeDtypeStruct((e, nt), jnp.float32),
        ],
    )(xf, W)


def _sc_select_body(sct_hbm, acc_hbm, cols, cdense, hist, totals,
                    cbits, cidx):
    c = lax.axis_index("c")
    sid = lax.axis_index("s")
    w = sid * 2 + c
    e0 = w * 2
    lanei = lax.iota(jnp.int32, 16)

    pltpu.sync_copy(sct_hbm.at[pl.ds(e0, 2)], cols)

    for j in range(2):
        e = e0 + j

        def zh(i, _):
            hist[pl.ds(i * 16, 16)] = jnp.zeros((16,), jnp.int32)
            return 0
        lax.fori_loop(0, 16 * _BINS // 16, zh, 0)

        def hb(i, _):
            bits = lax.bitcast_convert_type(cols[j, pl.ds(i * 16, 16)], jnp.int32)
            bin_ = lax.shift_right_logical(bits, 22)
            slot = lanei * _BINS + bin_
            plsc.addupdate_scatter(hist, [slot], jnp.ones((16,), jnp.int32))
            return 0
        lax.fori_loop(0, _NV, hb, 0)

        def fb(cc, _):
            acc = jnp.zeros((16,), jnp.int32)
            for l in range(16):
                acc = acc + hist[pl.ds(l * _BINS + cc * 16, 16)]
            totals[pl.ds(cc * 16, 16)] = acc
            return 0
        lax.fori_loop(0, _BINS // 16, fb, 0)

        def sb(k, carry):
            run, bstar, cntge = carry
            cc = _BINS // 16 - 1 - k
            t = totals[pl.ds(cc * 16, 16)]
            suf = lax.rev(plsc.cumsum(lax.rev(t, (0,))), (0,)) + run
            mask = suf >= _CAP
            npos = jnp.max(plsc.all_reduce_population_count(mask))
            found = (bstar < 0) & (npos > 0)
            local_b = cc * 16 + npos - 1
            cg = jnp.sum(jnp.where(lanei == (npos - 1), suf, 0))
            return (run + jnp.sum(t),
                    jnp.where(found, local_b, bstar),
                    jnp.where(found, cg, cntge))
        _, bstar, cntge = lax.fori_loop(
            0, _BINS // 16, sb,
            (jnp.int32(0), jnp.int32(-1), jnp.int32(0)))
        t_b = jnp.max(plsc.load_gather(totals, [jnp.full((16,), bstar)]))
        r0 = _CAP - (cntge - t_b)

        def xb(i, off):
            bits = lax.bitcast_convert_type(cols[j, pl.ds(i * 16, 16)], jnp.int32)
            m = lax.shift_right_logical(bits, 22) == bstar
            offc = jnp.minimum(off, 2048 - 16)
            plsc.store_compressed(cbits.at[pl.ds(offc, 16)], bits, mask=m)
            plsc.store_compressed(cidx.at[pl.ds(offc, 16)], lanei + i * 16, mask=m)
            return offc + jnp.max(plsc.all_reduce_population_count(m))
        ncand = lax.fori_loop(0, _NV, xb, jnp.int32(0))
        nv_c = (ncand + 15) >> 4

        def cnt_cand(pred):
            def body(i, acc):
                bb = cbits[pl.ds(i * 16, 16)]
                ii = cidx[pl.ds(i * 16, 16)]
                valid = (lanei + i * 16) < ncand
                m = pred(bb, ii) & valid
                return acc + jnp.max(plsc.all_reduce_population_count(m))
            return lax.fori_loop(0, nv_c, body, jnp.int32(0))

        base_bits = lax.shift_left(bstar, 22)

        def vb(_, lh):
            lo, hi = lh
            mid = lo + lax.shift_right_logical(hi - lo, 1)
            p = cnt_cand(lambda bb, ii: bb >= base_bits + mid) >= r0
            return jnp.where(p, mid, lo), jnp.where(p, hi, mid)
        lo22, _ = lax.fori_loop(0, 22, vb,
                                (jnp.int32(0), jnp.int32(1 << 22)))
        tau = base_bits + lo22
        r1 = r0 - cnt_cand(lambda bb, ii: bb > tau)

        def ib(_, lh):
            lo2, hi2 = lh
            mid = lo2 + lax.shift_right_logical(hi2 - lo2, 1)
            p = cnt_cand(lambda bb, ii: (bb == tau) & (ii < mid)) >= r1
            return jnp.where(p, lo2, mid), jnp.where(p, mid, hi2)
        _, ibound = lax.fori_loop(0, 15, ib,
                                  (jnp.int32(0), jnp.int32(_NT)))

        pw = lax.bitcast_convert_type(
            jnp.full((16,), (jnp.mod(e, 16) + 127) << 23, jnp.int32),
            jnp.float32)
        for row in range(32):
            def db(k, _):
                i = row * 64 + k
                bits = lax.bitcast_convert_type(cols[j, pl.ds(i * 16, 16)], jnp.int32)
                gidx = lanei + i * 16
                sel = (bits > tau) | ((bits == tau) & (gidx < ibound))
                contrib = jnp.where(sel, pw, jnp.float32(0.0))
                if j == 0:
                    cdense[row, pl.ds(k * 16, 16)] = contrib
                else:
                    cdense[row, pl.ds(k * 16, 16)] = (
                        cdense[row, pl.ds(k * 16, 16)] + contrib)
                return 0
            lax.fori_loop(0, 64, db, 0)

    pltpu.sync_copy(cdense, acc_hbm.at[w])


def _sc_combine_body(acc_hbm, scf_hbm, rw_hbm, ei_hbm,
                     accv, gi, gv, emaxv, eiv, rwv, sem):
    c = lax.axis_index("c")
    sid = lax.axis_index("s")
    w = sid * 2 + c
    base = w * 1024
    lanei = lax.iota(jnp.int32, 16)

    for t in range(32):
        pltpu.sync_copy(acc_hbm.at[t, w], accv.at[t])

    for row in range(8):
        def tb(k, _):
            i = row * 8 + k
            toks = base + i * 16 + lanei
            emax = jnp.full((16,), -1, jnp.int32)
            for g in range(4):
                a = accv[8 * g, pl.ds(i * 16, 16)]
                for t in range(8 * g + 1, 8 * g + 8):
                    a = a + accv[t, pl.ds(i * 16, 16)]
                ab = lax.bitcast_convert_type(a, jnp.int32)
                le = lax.shift_right_logical(ab, 23) - 127 + g * 16
                emax = jnp.maximum(emax, jnp.where(a > 0.0, le, -1))
            ei16 = jnp.maximum(emax, 0)
            gi[row, pl.ds(k * 16, 16)] = toks * _E + ei16
            emaxv[pl.ds(i * 16, 16)] = emax
            eiv[pl.ds(i * 16, 16)] = ei16
            return 0
        lax.fori_loop(0, 8, tb, 0)

    for row in range(8):
        pltpu.async_copy(scf_hbm.at[gi.at[row]], gv.at[row], sem).wait()

    for row in range(8):
        def rb(k, _):
            i = row * 8 + k
            m = emaxv[pl.ds(i * 16, 16)] >= 0
            val = gv[row, pl.ds(k * 16, 16)]
            rwv[pl.ds(i * 16, 16)] = jnp.where(m, val, jnp.float32(0.0))
            return 0
        lax.fori_loop(0, 8, rb, 0)

    pltpu.sync_copy(rwv, rw_hbm.at[pl.ds(base, 1024)])
    pltpu.sync_copy(eiv, ei_hbm.at[pl.ds(base, 1024)])


@functools.cache
def _sc_kernels():
    mesh = plsc.VectorSubcoreMesh(core_axis_name="c", subcore_axis_name="s")
    cparams = pltpu.CompilerParams(needs_layout_passes=False)
    sc_select = functools.partial(
        pl.kernel, mesh=mesh, compiler_params=cparams,
        out_type=jax.ShapeDtypeStruct((32, 32, 1024), jnp.float32),
        scratch_types=[
            pltpu.VMEM((2, _NT), jnp.float32),
            pltpu.VMEM((32, 1024), jnp.float32),
            pltpu.VMEM((16 * _BINS,), jnp.int32),
            pltpu.VMEM((_BINS,), jnp.int32),
            pltpu.VMEM((2048,), jnp.int32),
            pltpu.VMEM((2048,), jnp.int32),
        ],
    )(_sc_select_body)
    sc_combine = functools.partial(
        pl.kernel, mesh=mesh, compiler_params=cparams,
        out_type=[jax.ShapeDtypeStruct((_NT,), jnp.float32),
                  jax.ShapeDtypeStruct((_NT,), jnp.int32)],
        scratch_types=[
            pltpu.VMEM((32, 1024), jnp.float32),
            pltpu.VMEM((8, 128), jnp.int32),
            pltpu.VMEM((8, 128), jnp.float32),
            pltpu.VMEM((1024,), jnp.int32),
            pltpu.VMEM((1024,), jnp.int32),
            pltpu.VMEM((1024,), jnp.float32),
            pltpu.SemaphoreType.DMA,
        ],
    )(_sc_combine_body)
    return sc_select, sc_combine


def kernel(x, W):
    b, s, d = x.shape
    e = W.shape[0]
    nt = b * s
    xf = x.reshape(nt, d)
    sc_select, sc_combine = _sc_kernels()
    scores, scoresT = _tc_scores(xf, W, nt, e, s, b)
    acc = sc_select(scoresT)
    rw, ei = sc_combine(acc, scores.reshape(nt * e))
    return (rw.reshape(b, s, 1), ei.reshape(b, s, 1), scores.reshape(b, s, e))


# speedup vs baseline: 5.6147x; 1.0113x over previous
import functools

import jax
import jax.numpy as jnp
from jax import lax
from jax.experimental import pallas as pl
from jax.experimental.pallas import tpu as pltpu
from jax.experimental.pallas import tpu_sc as plsc

_CAPACITY_FACTOR = 1.25
_NT = 32768
_E = 64
_CAP = 640
_BINS = 512
_NV = _NT // 16


def _tc_body(x_ref, w_ref, sc_ref, sct_ref, *, nt, e, s, b, tblk, ch):
    i = pl.program_id(0)
    nb = pl.num_programs(0)
    lb = lax.dot_general(x_ref[...], w_ref[...], (((1,), (1,)), ((), ())),
                         preferred_element_type=jnp.float32)
    sc_ref[pl.ds(i * tblk, tblk), :] = lb

    @pl.when(i == nb - 1)
    def _final():
        nch_s = s // ch
        for bb in range(b):
            base = bb * s

            def mx_body(cc, m):
                blk = sc_ref[pl.ds(base + cc * ch, ch), :]
                return jnp.maximum(m, jnp.max(blk, axis=0, keepdims=True))
            m = lax.fori_loop(0, nch_s, mx_body,
                              jnp.full((1, e), -jnp.inf, jnp.float32))

            def ex_body(cc, z):
                idx = base + cc * ch
                eb = jnp.exp(sc_ref[pl.ds(idx, ch), :] - m)
                sc_ref[pl.ds(idx, ch), :] = eb
                return z + jnp.sum(eb, axis=0, keepdims=True)
            z = lax.fori_loop(0, nch_s, ex_body, jnp.zeros((1, e), jnp.float32))

            def nm_body(cc, carry):
                idx = base + cc * ch
                blk = sc_ref[pl.ds(idx, ch), :] / z
                sc_ref[pl.ds(idx, ch), :] = blk
                sct_ref[:, pl.ds(idx, ch)] = jnp.swapaxes(blk, 0, 1)
                return carry
            lax.fori_loop(0, nch_s, nm_body, 0)


def _tc_scores(xf, W, nt, e, s, b):
    tblk, ch = 512, 2048
    body = functools.partial(_tc_body, nt=nt, e=e, s=s, b=b, tblk=tblk, ch=ch)
    return pl.pallas_call(
        body,
        grid=(nt // tblk,),
        in_specs=[
            pl.BlockSpec((tblk, 768), lambda i: (i, 0)),
            pl.BlockSpec((e, 768), lambda i: (0, 0)),
        ],
        out_specs=[
            pl.BlockSpec((nt, e), lambda i: (0, 0)),
            pl.BlockSpec((e, nt), lambda i: (0, 0)),
        ],
        out_shape=[
            jax.ShapeDtypeStruct((nt, e), jnp.float32),
            jax.ShapeDtypeStruct((e, nt), jnp.float32),
        ],
    )(xf, W)


def _sc_select_body(sct_hbm, acc_hbm, cols, cdense, hist, totals,
                    cbits, cidx):
    c = lax.axis_index("c")
    sid = lax.axis_index("s")
    w = sid * 2 + c
    e0 = w * 2
    lanei = lax.iota(jnp.int32, 16)

    pltpu.sync_copy(sct_hbm.at[pl.ds(e0, 2)], cols)

    for j in range(2):
        e = e0 + j

        def zh(i, _):
            hist[pl.ds(i * 16, 16)] = jnp.zeros((16,), jnp.int32)
            return 0
        lax.fori_loop(0, 16 * _BINS // 16, zh, 0, unroll=4)

        def hb(i, _):
            bits = lax.bitcast_convert_type(cols[j, pl.ds(i * 16, 16)], jnp.int32)
            bin_ = lax.shift_right_logical(bits, 22)
            slot = lanei * _BINS + bin_
            plsc.addupdate_scatter(hist, [slot], jnp.ones((16,), jnp.int32))
            return 0
        lax.fori_loop(0, _NV, hb, 0, unroll=4)

        def fb(cc, _):
            acc = jnp.zeros((16,), jnp.int32)
            for l in range(16):
                acc = acc + hist[pl.ds(l * _BINS + cc * 16, 16)]
            totals[pl.ds(cc * 16, 16)] = acc
            return 0
        lax.fori_loop(0, _BINS // 16, fb, 0)

        def sb(k, carry):
            run, bstar, cntge = carry
            cc = _BINS // 16 - 1 - k
            t = totals[pl.ds(cc * 16, 16)]
            suf = lax.rev(plsc.cumsum(lax.rev(t, (0,))), (0,)) + run
            mask = suf >= _CAP
            npos = jnp.max(plsc.all_reduce_population_count(mask))
            found = (bstar < 0) & (npos > 0)
            local_b = cc * 16 + npos - 1
            cg = jnp.sum(jnp.where(lanei == (npos - 1), suf, 0))
            return (run + jnp.sum(t),
                    jnp.where(found, local_b, bstar),
                    jnp.where(found, cg, cntge))
        _, bstar, cntge = lax.fori_loop(
            0, _BINS // 16, sb,
            (jnp.int32(0), jnp.int32(-1), jnp.int32(0)))
        t_b = jnp.max(plsc.load_gather(totals, [jnp.full((16,), bstar)]))
        r0 = _CAP - (cntge - t_b)

        def xb(i, off):
            bits = lax.bitcast_convert_type(cols[j, pl.ds(i * 16, 16)], jnp.int32)
            m = lax.shift_right_logical(bits, 22) == bstar
            offc = jnp.minimum(off, 2048 - 16)
            plsc.store_compressed(cbits.at[pl.ds(offc, 16)], bits, mask=m)
            plsc.store_compressed(cidx.at[pl.ds(offc, 16)], lanei + i * 16, mask=m)
            return offc + jnp.max(plsc.all_reduce_population_count(m))
        ncand = lax.fori_loop(0, _NV, xb, jnp.int32(0), unroll=4)
        nv_c = (ncand + 15) >> 4

        def cnt_cand(pred):
            def body(i, acc):
                bb = cbits[pl.ds(i * 16, 16)]
                ii = cidx[pl.ds(i * 16, 16)]
                valid = (lanei + i * 16) < ncand
                m = pred(bb, ii) & valid
                return acc + jnp.max(plsc.all_reduce_population_count(m))
            return lax.fori_loop(0, nv_c, body, jnp.int32(0))

        base_bits = lax.shift_left(bstar, 22)

        def vb(_, lh):
            lo, hi = lh
            mid = lo + lax.shift_right_logical(hi - lo, 1)
            p = cnt_cand(lambda bb, ii: bb >= base_bits + mid) >= r0
            return jnp.where(p, mid, lo), jnp.where(p, hi, mid)
        lo22, _ = lax.fori_loop(0, 22, vb,
                                (jnp.int32(0), jnp.int32(1 << 22)))
        tau = base_bits + lo22
        r1 = r0 - cnt_cand(lambda bb, ii: bb > tau)

        def ib(_, lh):
            lo2, hi2 = lh
            mid = lo2 + lax.shift_right_logical(hi2 - lo2, 1)
            p = cnt_cand(lambda bb, ii: (bb == tau) & (ii < mid)) >= r1
            return jnp.where(p, lo2, mid), jnp.where(p, mid, hi2)
        _, ibound = lax.fori_loop(0, 15, ib,
                                  (jnp.int32(0), jnp.int32(_NT)))

        pw = lax.bitcast_convert_type(
            jnp.full((16,), (jnp.mod(e, 16) + 127) << 23, jnp.int32),
            jnp.float32)
        for row in range(32):
            def db(k, _):
                i = row * 64 + k
                bits = lax.bitcast_convert_type(cols[j, pl.ds(i * 16, 16)], jnp.int32)
                gidx = lanei + i * 16
                sel = (bits > tau) | ((bits == tau) & (gidx < ibound))
                contrib = jnp.where(sel, pw, jnp.float32(0.0))
                if j == 0:
                    cdense[row, pl.ds(k * 16, 16)] = contrib
                else:
                    cdense[row, pl.ds(k * 16, 16)] = (
                        cdense[row, pl.ds(k * 16, 16)] + contrib)
                return 0
            lax.fori_loop(0, 64, db, 0)

    pltpu.sync_copy(cdense, acc_hbm.at[w])


def _sc_combine_body(acc_hbm, scf_hbm, rw_hbm, ei_hbm,
                     accv, gi, gv, emaxv, eiv, rwv, sem):
    c = lax.axis_index("c")
    sid = lax.axis_index("s")
    w = sid * 2 + c
    base = w * 1024
    lanei = lax.iota(jnp.int32, 16)

    for t in range(32):
        pltpu.sync_copy(acc_hbm.at[t, w], accv.at[t])

    for row in range(8):
        def tb(k, _):
            i = row * 8 + k
            toks = base + i * 16 + lanei
            emax = jnp.full((16,), -1, jnp.int32)
            for g in range(4):
                a = accv[8 * g, pl.ds(i * 16, 16)]
                for t in range(8 * g + 1, 8 * g + 8):
                    a = a + accv[t, pl.ds(i * 16, 16)]
                ab = lax.bitcast_convert_type(a, jnp.int32)
                le = lax.shift_right_logical(ab, 23) - 127 + g * 16
                emax = jnp.maximum(emax, jnp.where(a > 0.0, le, -1))
            ei16 = jnp.maximum(emax, 0)
            gi[row, pl.ds(k * 16, 16)] = toks * _E + ei16
            emaxv[pl.ds(i * 16, 16)] = emax
            eiv[pl.ds(i * 16, 16)] = ei16
            return 0
        lax.fori_loop(0, 8, tb, 0)

    for row in range(8):
        pltpu.async_copy(scf_hbm.at[gi.at[row]], gv.at[row], sem).wait()

    for row in range(8):
        def rb(k, _):
            i = row * 8 + k
            m = emaxv[pl.ds(i * 16, 16)] >= 0
            val = gv[row, pl.ds(k * 16, 16)]
            rwv[pl.ds(i * 16, 16)] = jnp.where(m, val, jnp.float32(0.0))
            return 0
        lax.fori_loop(0, 8, rb, 0)

    pltpu.sync_copy(rwv, rw_hbm.at[pl.ds(base, 1024)])
    pltpu.sync_copy(eiv, ei_hbm.at[pl.ds(base, 1024)])


@functools.cache
def _sc_kernels():
    mesh = plsc.VectorSubcoreMesh(core_axis_name="c", subcore_axis_name="s")
    cparams = pltpu.CompilerParams(needs_layout_passes=False)
    sc_select = functools.partial(
        pl.kernel, mesh=mesh, compiler_params=cparams,
        out_type=jax.ShapeDtypeStruct((32, 32, 1024), jnp.float32),
        scratch_types=[
            pltpu.VMEM((2, _NT), jnp.float32),
            pltpu.VMEM((32, 1024), jnp.float32),
            pltpu.VMEM((16 * _BINS,), jnp.int32),
            pltpu.VMEM((_BINS,), jnp.int32),
            pltpu.VMEM((2048,), jnp.int32),
            pltpu.VMEM((2048,), jnp.int32),
        ],
    )(_sc_select_body)
    sc_combine = functools.partial(
        pl.kernel, mesh=mesh, compiler_params=cparams,
        out_type=[jax.ShapeDtypeStruct((_NT,), jnp.float32),
                  jax.ShapeDtypeStruct((_NT,), jnp.int32)],
        scratch_types=[
            pltpu.VMEM((32, 1024), jnp.float32),
            pltpu.VMEM((8, 128), jnp.int32),
            pltpu.VMEM((8, 128), jnp.float32),
            pltpu.VMEM((1024,), jnp.int32),
            pltpu.VMEM((1024,), jnp.int32),
            pltpu.VMEM((1024,), jnp.float32),
            pltpu.SemaphoreType.DMA,
        ],
    )(_sc_combine_body)
    return sc_select, sc_combine


def kernel(x, W):
    b, s, d = x.shape
    e = W.shape[0]
    nt = b * s
    xf = x.reshape(nt, d)
    sc_select, sc_combine = _sc_kernels()
    scores, scoresT = _tc_scores(xf, W, nt, e, s, b)
    acc = sc_select(scoresT)
    rw, ei = sc_combine(acc, scores.reshape(nt * e))
    return (rw.reshape(b, s, 1), ei.reshape(b, s, 1), scores.reshape(b, s, e))
